# baseline (speedup 1.0000x reference)
import functools

import jax
import jax.numpy as jnp
from jax import lax
from jax.experimental import pallas as pl
from jax.experimental.pallas import tpu as pltpu
from jax.experimental.pallas import tpu_sc as plsc

N = 10000
E = 320000
D = 128
H = 128
LH = 128
C = 10
G = 64

NC = 2
NS = 16
NW = NC * NS
EPW = E // NW
K = 128
CH = 80
PADW = CH * K - EPW
NP = N + 8
NZ = 10
RPS = N // NZ

NQ = 1

NB = 5
R = N // NB

_sc_mesh = plsc.VectorSubcoreMesh(core_axis_name="c", subcore_axis_name="s")


def _zero_rows(buf, nrows):
    def _zrow(i, carry):
        for j in range(H // 16):
            buf[i, pl.ds(j * 16, 16)] = jnp.zeros((16,), jnp.float32)
        return carry

    lax.fori_loop(0, nrows, _zrow, 0)


def _zero_shared(zbuf, shared, s):
    @pl.when(s < NZ)
    def _zero():
        for t in range(RPS // K):
            pltpu.sync_copy(zbuf.at[pl.ds(0, K)],
                            shared.at[pl.ds(s * RPS + t * K, K)])
        rem = RPS - (RPS // K) * K
        pltpu.sync_copy(zbuf.at[pl.ds(0, rem)],
                        shared.at[pl.ds(s * RPS + (RPS // K) * K, rem)])


def _copy_out(shared, out_hbm, c, s):
    @pl.when(s < NZ)
    def _copy():
        pltpu.sync_copy(shared.at[pl.ds(s * RPS, RPS)],
                        out_hbm.at[c, pl.ds(s * RPS, RPS)])


@functools.partial(
    pl.kernel,
    out_type=jax.ShapeDtypeStruct((NC, N, H), jnp.float32),
    mesh=_sc_mesh,
    scratch_types=[
        pltpu.VMEM((CH, K), jnp.int32),
        pltpu.VMEM((K, H), jnp.float32),
        pltpu.VMEM_SHARED((NP, H), jnp.float32),
    ],
)
def _deg_kernel(dst_hbm, out_hbm, dst_v, ones_v, spdeg):
    c = lax.axis_index("c")
    s = lax.axis_index("s")
    w = c * NS + s

    _zero_rows(ones_v, K)
    _zero_shared(ones_v, spdeg, s)

    def _orow(i, carry):
        ones_v[i, pl.ds(0, 16)] = jnp.ones((16,), jnp.float32)
        return carry

    lax.fori_loop(0, K, _orow, 0)
    plsc.subcore_barrier()

    pltpu.sync_copy(dst_hbm.at[w], dst_v)

    def _body(i, carry):
        pltpu.sync_copy(ones_v, spdeg.at[dst_v.at[i]], add=True)
        return carry

    lax.fori_loop(0, CH, _body, 0)
    plsc.subcore_barrier()

    _copy_out(spdeg, out_hbm, c, s)


@functools.partial(
    pl.kernel,
    out_type=jax.ShapeDtypeStruct((NC, N, H), jnp.float32),
    mesh=_sc_mesh,
    scratch_types=[
        pltpu.VMEM((CH, K), jnp.int32),
        pltpu.VMEM((8, K), jnp.int32),
        pltpu.VMEM((8, K), jnp.int32),
        pltpu.VMEM((2 * K, H), jnp.float32),
        pltpu.VMEM_SHARED((NP, H), jnp.float32),
        [pltpu.SemaphoreType.DMA] * NQ,
        [pltpu.SemaphoreType.DMA] * NQ,
        pltpu.SemaphoreType.DMA,
        pltpu.SemaphoreType.DMA,
    ],
)
def _scatter_kernel(hn_hbm, pidx_hbm, out_hbm,
                    pk_v, sring, dring, rows_v, spagg, g0, g1, s0, s1):
    c = lax.axis_index("c")
    s = lax.axis_index("s")
    w = c * NS + s
    QR = K // NQ

    def _gather(slot, gsems):
        for q in range(NQ):
            pltpu.async_copy(
                hn_hbm.at[sring.at[slot, pl.ds(q * QR, QR)]],
                rows_v.at[pl.ds(slot * K + q * QR, QR)], gsems[q])

    def _wait_gather(gsems):
        for q in range(NQ):
            pltpu.make_async_copy(hn_hbm.at[pl.ds(0, QR)],
                                  rows_v.at[pl.ds(0, QR)], gsems[q]).wait()

    def _wait64k(sem):
        pltpu.make_async_copy(hn_hbm.at[pl.ds(0, K)],
                              rows_v.at[pl.ds(0, K)], sem).wait()

    def _unpack_src(j, slot):
        for tt in range(K // 16):
            v = pk_v[j, pl.ds(tt * 16, 16)]
            sring[slot, pl.ds(tt * 16, 16)] = jnp.bitwise_and(v, 0xFFFF)

    def _unpack_dst(j, slot):
        for tt in range(K // 16):
            v = pk_v[j, pl.ds(tt * 16, 16)]
            dring[slot, pl.ds(tt * 16, 16)] = lax.shift_right_logical(v, 16)

    _zero_rows(rows_v, K)
    _zero_shared(rows_v, spagg, s)

    pltpu.sync_copy(pidx_hbm.at[w], pk_v)
    _unpack_src(0, 0)
    _unpack_dst(0, 0)
    _unpack_src(1, 1)
    _unpack_dst(1, 1)
    plsc.subcore_barrier()

    _gather(0, g0)
    _gather(1, g1)

    def _body(t, carry):
        g = 2 * t
        _wait_gather(g0)
        pltpu.async_copy(rows_v.at[pl.ds(0, K)], spagg.at[dring.at[0]], s0,
                         add=True)
        _wait_gather(g1)
        pltpu.async_copy(rows_v.at[pl.ds(K, K)], spagg.at[dring.at[1]], s1,
                         add=True)
        _unpack_src(g + 2, 0)
        _wait64k(s0)
        _unpack_dst(g + 2, 0)
        _gather(0, g0)
        _unpack_src(g + 3, 1)
        _wait64k(s1)
        _unpack_dst(g + 3, 1)
        _gather(1, g1)
        return carry

    lax.fori_loop(0, (CH - 2) // 2, _body, 0)

    _wait_gather(g0)
    pltpu.sync_copy(rows_v.at[pl.ds(0, K)], spagg.at[dring.at[0]], add=True)
    _wait_gather(g1)
    pltpu.sync_copy(rows_v.at[pl.ds(K, K)], spagg.at[dring.at[1]], add=True)
    plsc.subcore_barrier()

    _copy_out(spagg, out_hbm, c, s)


def _mm1_body(deg_ref, x_ref, w_ref, out_ref):
    d = deg_ref[0, :, 0:1] + deg_ref[1, :, 0:1] + 1.0
    dinv = lax.rsqrt(d)
    h = jnp.dot(x_ref[...], w_ref[...], preferred_element_type=jnp.float32)
    out_ref[...] = dinv * h


_mm1 = pl.pallas_call(
    _mm1_body,
    grid=(NB,),
    in_specs=[
        pl.BlockSpec((NC, R, H), lambda i: (0, i, 0)),
        pl.BlockSpec((R, D), lambda i: (i, 0)),
        pl.BlockSpec((D, H), lambda i: (0, 0)),
    ],
    out_specs=pl.BlockSpec((R, H), lambda i: (i, 0)),
    out_shape=jax.ShapeDtypeStruct((N, H), jnp.float32),
)


def _mm2_body(deg_ref, p_ref, hn_ref, b_ref, w_ref, out_ref):
    d = deg_ref[0, :, 0:1] + deg_ref[1, :, 0:1] + 1.0
    dinv = lax.rsqrt(d)
    a = jnp.maximum(dinv * (p_ref[0] + p_ref[1] + hn_ref[...]) + b_ref[...],
                    0.0)
    out_ref[...] = dinv * jnp.dot(a, w_ref[...],
                                  preferred_element_type=jnp.float32)


_mm2 = pl.pallas_call(
    _mm2_body,
    grid=(NB,),
    in_specs=[
        pl.BlockSpec((NC, R, H), lambda i: (0, i, 0)),
        pl.BlockSpec((NC, R, H), lambda i: (0, i, 0)),
        pl.BlockSpec((R, H), lambda i: (i, 0)),
        pl.BlockSpec((1, H), lambda i: (0, 0)),
        pl.BlockSpec((H, H), lambda i: (0, 0)),
    ],
    out_specs=pl.BlockSpec((R, H), lambda i: (i, 0)),
    out_shape=jax.ShapeDtypeStruct((N, H), jnp.float32),
)


def _head_body(deg_ref, p_ref, hn_ref, b2_ref, batch_ref, wih_ref, bih_ref,
               bhh_ref, wfc_ref, bfc_ref, out_ref, acc_ref, cnt_ref):
    i = pl.program_id(0)

    @pl.when(i == 0)
    def _init():
        acc_ref[...] = jnp.zeros_like(acc_ref)
        cnt_ref[...] = jnp.zeros_like(cnt_ref)

    d = deg_ref[0, :, 0:1] + deg_ref[1, :, 0:1] + 1.0
    dinv = lax.rsqrt(d)
    h2 = jnp.maximum(dinv * (p_ref[0] + p_ref[1] + hn_ref[...]) + b2_ref[...],
                     0.0)
    b = batch_ref[0]
    gid = lax.broadcasted_iota(jnp.int32, (G, 1), 0)
    mask = (b == gid).astype(jnp.float32)
    acc_ref[...] += jnp.dot(mask, h2, preferred_element_type=jnp.float32)
    cnt_ref[...] += jnp.sum(mask, axis=1, keepdims=True)

    @pl.when(i == NB - 1)
    def _final():
        pooled = acc_ref[...] / jnp.maximum(cnt_ref[...], 1.0)
        gates = lax.dot_general(
            pooled, wih_ref[...], (((1,), (1,)), ((), ())),
            preferred_element_type=jnp.float32,
        ) + bih_ref[...] + bhh_ref[...]
        gi = gates[:, 0:LH]
        gg = gates[:, 2 * LH:3 * LH]
        go = gates[:, 3 * LH:4 * LH]
        cc = jax.nn.sigmoid(gi) * jnp.tanh(gg)
        hh = jax.nn.sigmoid(go) * jnp.tanh(cc)
        out_ref[...] = lax.dot_general(
            hh, wfc_ref[...], (((1,), (1,)), ((), ())),
            preferred_element_type=jnp.float32,
        ) + bfc_ref[...]


_head = pl.pallas_call(
    _head_body,
    grid=(NB,),
    in_specs=[
        pl.BlockSpec((NC, R, H), lambda i: (0, i, 0)),
        pl.BlockSpec((NC, R, H), lambda i: (0, i, 0)),
        pl.BlockSpec((R, H), lambda i: (i, 0)),
        pl.BlockSpec((1, H), lambda i: (0, 0)),
        pl.BlockSpec((1, 1, R), lambda i: (i, 0, 0)),
        pl.BlockSpec((4 * LH, H), lambda i: (0, 0)),
        pl.BlockSpec((1, 4 * LH), lambda i: (0, 0)),
        pl.BlockSpec((1, 4 * LH), lambda i: (0, 0)),
        pl.BlockSpec((C, LH), lambda i: (0, 0)),
        pl.BlockSpec((1, C), lambda i: (0, 0)),
    ],
    out_specs=pl.BlockSpec((G, C), lambda i: (0, 0)),
    out_shape=jax.ShapeDtypeStruct((G, C), jnp.float32),
    scratch_shapes=[
        pltpu.VMEM((G, LH), jnp.float32),
        pltpu.VMEM((G, 1), jnp.float32),
    ],
)


def _pad_edges(idx, fill):
    per_w = idx.reshape(NW, EPW)
    pad = jnp.full((NW, PADW), fill, dtype=idx.dtype)
    return jnp.concatenate([per_w, pad], axis=1).reshape(NW, CH, K)


def kernel(x, edge_index, batch, W1, b1, W2, b2, W_ih, W_hh, b_ih, b_hh,
           W_fc, b_fc):
    src = _pad_edges(edge_index[0], 0)
    dst = _pad_edges(edge_index[1], N)
    pidx = jnp.bitwise_or(src, jnp.left_shift(dst, 16))

    degp = _deg_kernel(dst)
    hn1 = _mm1(degp, x, W1)
    part1 = _scatter_kernel(hn1, pidx)
    hn2 = _mm2(degp, part1, hn1, b1.reshape(1, H), W2)
    part2 = _scatter_kernel(hn2, pidx)
    out = _head(degp, part2, hn2, b2.reshape(1, H),
                batch.reshape(NB, 1, R), W_ih, b_ih.reshape(1, 4 * LH),
                b_hh.reshape(1, 4 * LH), W_fc, b_fc.reshape(1, C))
    return out


# speedup vs baseline: 1.0397x; 1.0397x over previous
import functools

import jax
import jax.numpy as jnp
from jax import lax
from jax.experimental import pallas as pl
from jax.experimental.pallas import tpu as pltpu
from jax.experimental.pallas import tpu_sc as plsc

N = 10000
E = 320000
D = 128
H = 128
LH = 128
C = 10
G = 64

NC = 2
NS = 16
NW = NC * NS
EPW = E // NW
K = 128
CH = 80
PADW = CH * K - EPW
NP = N + 8
NZ = 10
RPS = N // NZ

NQ = 8

NB = 5
R = N // NB

_sc_mesh = plsc.VectorSubcoreMesh(core_axis_name="c", subcore_axis_name="s")


def _zero_rows(buf, nrows):
    def _zrow(i, carry):
        for j in range(H // 16):
            buf[i, pl.ds(j * 16, 16)] = jnp.zeros((16,), jnp.float32)
        return carry

    lax.fori_loop(0, nrows, _zrow, 0)


def _zero_shared(zbuf, shared, s):
    @pl.when(s < NZ)
    def _zero():
        for t in range(RPS // K):
            pltpu.sync_copy(zbuf.at[pl.ds(0, K)],
                            shared.at[pl.ds(s * RPS + t * K, K)])
        rem = RPS - (RPS // K) * K
        pltpu.sync_copy(zbuf.at[pl.ds(0, rem)],
                        shared.at[pl.ds(s * RPS + (RPS // K) * K, rem)])


def _copy_out(shared, out_hbm, c, s):
    @pl.when(s < NZ)
    def _copy():
        pltpu.sync_copy(shared.at[pl.ds(s * RPS, RPS)],
                        out_hbm.at[c, pl.ds(s * RPS, RPS)])


@functools.partial(
    pl.kernel,
    out_type=jax.ShapeDtypeStruct((NC, N, H), jnp.float32),
    mesh=_sc_mesh,
    scratch_types=[
        pltpu.VMEM((CH, K), jnp.int32),
        pltpu.VMEM((K, H), jnp.float32),
        pltpu.VMEM_SHARED((NP, H), jnp.float32),
    ],
)
def _deg_kernel(dst_hbm, out_hbm, dst_v, ones_v, spdeg):
    c = lax.axis_index("c")
    s = lax.axis_index("s")
    w = c * NS + s

    _zero_rows(ones_v, K)
    _zero_shared(ones_v, spdeg, s)

    def _orow(i, carry):
        ones_v[i, pl.ds(0, 16)] = jnp.ones((16,), jnp.float32)
        return carry

    lax.fori_loop(0, K, _orow, 0)
    plsc.subcore_barrier()

    pltpu.sync_copy(dst_hbm.at[w], dst_v)

    def _body(i, carry):
        pltpu.sync_copy(ones_v, spdeg.at[dst_v.at[i]], add=True)
        return carry

    lax.fori_loop(0, CH, _body, 0)
    plsc.subcore_barrier()

    _copy_out(spdeg, out_hbm, c, s)


@functools.partial(
    pl.kernel,
    out_type=jax.ShapeDtypeStruct((NC, N, H), jnp.float32),
    mesh=_sc_mesh,
    scratch_types=[
        pltpu.VMEM((CH, K), jnp.int32),
        pltpu.VMEM((8, K), jnp.int32),
        pltpu.VMEM((8, K), jnp.int32),
        pltpu.VMEM((2 * K, H), jnp.float32),
        pltpu.VMEM_SHARED((NP, H), jnp.float32),
        [pltpu.SemaphoreType.DMA] * NQ,
        [pltpu.SemaphoreType.DMA] * NQ,
        pltpu.SemaphoreType.DMA,
        pltpu.SemaphoreType.DMA,
    ],
)
def _scatter_kernel(hn_hbm, pidx_hbm, out_hbm,
                    pk_v, sring, dring, rows_v, spagg, g0, g1, s0, s1):
    c = lax.axis_index("c")
    s = lax.axis_index("s")
    w = c * NS + s
    QR = K // NQ

    def _gather(slot, gsems):
        for q in range(NQ):
            pltpu.async_copy(
                hn_hbm.at[sring.at[slot, pl.ds(q * QR, QR)]],
                rows_v.at[pl.ds(slot * K + q * QR, QR)], gsems[q])

    def _wait_gather(gsems):
        for q in range(NQ):
            pltpu.make_async_copy(hn_hbm.at[pl.ds(0, QR)],
                                  rows_v.at[pl.ds(0, QR)], gsems[q]).wait()

    def _wait64k(sem):
        pltpu.make_async_copy(hn_hbm.at[pl.ds(0, K)],
                              rows_v.at[pl.ds(0, K)], sem).wait()

    def _unpack_src(j, slot):
        for tt in range(K // 16):
            v = pk_v[j, pl.ds(tt * 16, 16)]
            sring[slot, pl.ds(tt * 16, 16)] = jnp.bitwise_and(v, 0xFFFF)

    def _unpack_dst(j, slot):
        for tt in range(K // 16):
            v = pk_v[j, pl.ds(tt * 16, 16)]
            dring[slot, pl.ds(tt * 16, 16)] = lax.shift_right_logical(v, 16)

    _zero_rows(rows_v, K)
    _zero_shared(rows_v, spagg, s)

    pltpu.sync_copy(pidx_hbm.at[w], pk_v)
    _unpack_src(0, 0)
    _unpack_dst(0, 0)
    _unpack_src(1, 1)
    _unpack_dst(1, 1)
    plsc.subcore_barrier()

    _gather(0, g0)
    _gather(1, g1)

    def _body(t, carry):
        g = 2 * t
        _wait_gather(g0)
        pltpu.async_copy(rows_v.at[pl.ds(0, K)], spagg.at[dring.at[0]], s0,
                         add=True)
        _wait_gather(g1)
        pltpu.async_copy(rows_v.at[pl.ds(K, K)], spagg.at[dring.at[1]], s1,
                         add=True)
        _unpack_src(g + 2, 0)
        _wait64k(s0)
        _unpack_dst(g + 2, 0)
        _gather(0, g0)
        _unpack_src(g + 3, 1)
        _wait64k(s1)
        _unpack_dst(g + 3, 1)
        _gather(1, g1)
        return carry

    lax.fori_loop(0, (CH - 2) // 2, _body, 0)

    _wait_gather(g0)
    pltpu.sync_copy(rows_v.at[pl.ds(0, K)], spagg.at[dring.at[0]], add=True)
    _wait_gather(g1)
    pltpu.sync_copy(rows_v.at[pl.ds(K, K)], spagg.at[dring.at[1]], add=True)
    plsc.subcore_barrier()

    _copy_out(spagg, out_hbm, c, s)


def _mm1_body(deg_ref, x_ref, w_ref, out_ref):
    d = deg_ref[0, :, 0:1] + deg_ref[1, :, 0:1] + 1.0
    dinv = lax.rsqrt(d)
    h = jnp.dot(x_ref[...], w_ref[...], preferred_element_type=jnp.float32)
    out_ref[...] = dinv * h


_mm1 = pl.pallas_call(
    _mm1_body,
    grid=(NB,),
    in_specs=[
        pl.BlockSpec((NC, R, H), lambda i: (0, i, 0)),
        pl.BlockSpec((R, D), lambda i: (i, 0)),
        pl.BlockSpec((D, H), lambda i: (0, 0)),
    ],
    out_specs=pl.BlockSpec((R, H), lambda i: (i, 0)),
    out_shape=jax.ShapeDtypeStruct((N, H), jnp.float32),
)


def _mm2_body(deg_ref, p_ref, hn_ref, b_ref, w_ref, out_ref):
    d = deg_ref[0, :, 0:1] + deg_ref[1, :, 0:1] + 1.0
    dinv = lax.rsqrt(d)
    a = jnp.maximum(dinv * (p_ref[0] + p_ref[1] + hn_ref[...]) + b_ref[...],
                    0.0)
    out_ref[...] = dinv * jnp.dot(a, w_ref[...],
                                  preferred_element_type=jnp.float32)


_mm2 = pl.pallas_call(
    _mm2_body,
    grid=(NB,),
    in_specs=[
        pl.BlockSpec((NC, R, H), lambda i: (0, i, 0)),
        pl.BlockSpec((NC, R, H), lambda i: (0, i, 0)),
        pl.BlockSpec((R, H), lambda i: (i, 0)),
        pl.BlockSpec((1, H), lambda i: (0, 0)),
        pl.BlockSpec((H, H), lambda i: (0, 0)),
    ],
    out_specs=pl.BlockSpec((R, H), lambda i: (i, 0)),
    out_shape=jax.ShapeDtypeStruct((N, H), jnp.float32),
)


def _head_body(deg_ref, p_ref, hn_ref, b2_ref, batch_ref, wih_ref, bih_ref,
               bhh_ref, wfc_ref, bfc_ref, out_ref, acc_ref, cnt_ref):
    i = pl.program_id(0)

    @pl.when(i == 0)
    def _init():
        acc_ref[...] = jnp.zeros_like(acc_ref)
        cnt_ref[...] = jnp.zeros_like(cnt_ref)

    d = deg_ref[0, :, 0:1] + deg_ref[1, :, 0:1] + 1.0
    dinv = lax.rsqrt(d)
    h2 = jnp.maximum(dinv * (p_ref[0] + p_ref[1] + hn_ref[...]) + b2_ref[...],
                     0.0)
    b = batch_ref[0]
    gid = lax.broadcasted_iota(jnp.int32, (G, 1), 0)
    mask = (b == gid).astype(jnp.float32)
    acc_ref[...] += jnp.dot(mask, h2, preferred_element_type=jnp.float32)
    cnt_ref[...] += jnp.sum(mask, axis=1, keepdims=True)

    @pl.when(i == NB - 1)
    def _final():
        pooled = acc_ref[...] / jnp.maximum(cnt_ref[...], 1.0)
        gates = lax.dot_general(
            pooled, wih_ref[...], (((1,), (1,)), ((), ())),
            preferred_element_type=jnp.float32,
        ) + bih_ref[...] + bhh_ref[...]
        gi = gates[:, 0:LH]
        gg = gates[:, 2 * LH:3 * LH]
        go = gates[:, 3 * LH:4 * LH]
        cc = jax.nn.sigmoid(gi) * jnp.tanh(gg)
        hh = jax.nn.sigmoid(go) * jnp.tanh(cc)
        out_ref[...] = lax.dot_general(
            hh, wfc_ref[...], (((1,), (1,)), ((), ())),
            preferred_element_type=jnp.float32,
        ) + bfc_ref[...]


_head = pl.pallas_call(
    _head_body,
    grid=(NB,),
    in_specs=[
        pl.BlockSpec((NC, R, H), lambda i: (0, i, 0)),
        pl.BlockSpec((NC, R, H), lambda i: (0, i, 0)),
        pl.BlockSpec((R, H), lambda i: (i, 0)),
        pl.BlockSpec((1, H), lambda i: (0, 0)),
        pl.BlockSpec((1, 1, R), lambda i: (i, 0, 0)),
        pl.BlockSpec((4 * LH, H), lambda i: (0, 0)),
        pl.BlockSpec((1, 4 * LH), lambda i: (0, 0)),
        pl.BlockSpec((1, 4 * LH), lambda i: (0, 0)),
        pl.BlockSpec((C, LH), lambda i: (0, 0)),
        pl.BlockSpec((1, C), lambda i: (0, 0)),
    ],
    out_specs=pl.BlockSpec((G, C), lambda i: (0, 0)),
    out_shape=jax.ShapeDtypeStruct((G, C), jnp.float32),
    scratch_shapes=[
        pltpu.VMEM((G, LH), jnp.float32),
        pltpu.VMEM((G, 1), jnp.float32),
    ],
)


def _pad_edges(idx, fill):
    per_w = idx.reshape(NW, EPW)
    pad = jnp.full((NW, PADW), fill, dtype=idx.dtype)
    return jnp.concatenate([per_w, pad], axis=1).reshape(NW, CH, K)


def kernel(x, edge_index, batch, W1, b1, W2, b2, W_ih, W_hh, b_ih, b_hh,
           W_fc, b_fc):
    src = _pad_edges(edge_index[0], 0)
    dst = _pad_edges(edge_index[1], N)
    pidx = jnp.bitwise_or(src, jnp.left_shift(dst, 16))

    degp = _deg_kernel(dst)
    hn1 = _mm1(degp, x, W1)
    part1 = _scatter_kernel(hn1, pidx)
    hn2 = _mm2(degp, part1, hn1, b1.reshape(1, H), W2)
    part2 = _scatter_kernel(hn2, pidx)
    out = _head(degp, part2, hn2, b2.reshape(1, H),
                batch.reshape(NB, 1, R), W_ih, b_ih.reshape(1, 4 * LH),
                b_hh.reshape(1, 4 * LH), W_fc, b_fc.reshape(1, C))
    return out


# speedup vs baseline: 1.0414x; 1.0017x over previous
import functools

import jax
import jax.numpy as jnp
from jax import lax
from jax.experimental import pallas as pl
from jax.experimental.pallas import tpu as pltpu
from jax.experimental.pallas import tpu_sc as plsc

N = 10000
E = 320000
D = 128
H = 128
LH = 128
C = 10
G = 64

NC = 2
NS = 16
NW = NC * NS
EPW = E // NW
K = 128
CH = 80
PADW = CH * K - EPW
NP = N + 8
NZ = 10
RPS = N // NZ

NQ = 8

NB = 5
R = N // NB

_sc_mesh = plsc.VectorSubcoreMesh(core_axis_name="c", subcore_axis_name="s")


def _zero_rows(buf, nrows):
    def _zrow(i, carry):
        for j in range(H // 16):
            buf[i, pl.ds(j * 16, 16)] = jnp.zeros((16,), jnp.float32)
        return carry

    lax.fori_loop(0, nrows, _zrow, 0)


def _zero_shared(zbuf, shared, s):
    @pl.when(s < NZ)
    def _zero():
        for t in range(RPS // K):
            pltpu.sync_copy(zbuf.at[pl.ds(0, K)],
                            shared.at[pl.ds(s * RPS + t * K, K)])
        rem = RPS - (RPS // K) * K
        pltpu.sync_copy(zbuf.at[pl.ds(0, rem)],
                        shared.at[pl.ds(s * RPS + (RPS // K) * K, rem)])


def _copy_out(shared, out_hbm, c, s):
    @pl.when(s < NZ)
    def _copy():
        pltpu.sync_copy(shared.at[pl.ds(s * RPS, RPS)],
                        out_hbm.at[c, pl.ds(s * RPS, RPS)])


ND = 8


@functools.partial(
    pl.kernel,
    out_type=jax.ShapeDtypeStruct((NC, N, H), jnp.float32),
    mesh=_sc_mesh,
    scratch_types=[
        pltpu.VMEM((CH, K), jnp.int32),
        pltpu.VMEM((K, H), jnp.float32),
        pltpu.VMEM_SHARED((NP, H), jnp.float32),
        [pltpu.SemaphoreType.DMA] * ND,
    ],
)
def _deg_kernel(dst_hbm, out_hbm, dst_v, ones_v, spdeg, dsems):
    c = lax.axis_index("c")
    s = lax.axis_index("s")
    w = c * NS + s

    _zero_rows(ones_v, K)
    _zero_shared(ones_v, spdeg, s)

    def _orow(i, carry):
        ones_v[i, pl.ds(0, 16)] = jnp.ones((16,), jnp.float32)
        return carry

    lax.fori_loop(0, K, _orow, 0)
    plsc.subcore_barrier()

    pltpu.sync_copy(dst_hbm.at[w], dst_v)

    def _body(t, carry):
        for u in range(ND):
            pltpu.async_copy(ones_v, spdeg.at[dst_v.at[t * ND + u]],
                             dsems[u], add=True)
        for u in range(ND):
            pltpu.make_async_copy(out_hbm.at[0, pl.ds(0, K)], ones_v,
                                  dsems[u]).wait()
        return carry

    lax.fori_loop(0, CH // ND, _body, 0)
    plsc.subcore_barrier()

    _copy_out(spdeg, out_hbm, c, s)


@functools.partial(
    pl.kernel,
    out_type=jax.ShapeDtypeStruct((NC, N, H), jnp.float32),
    mesh=_sc_mesh,
    scratch_types=[
        pltpu.VMEM((CH, K), jnp.int32),
        pltpu.VMEM((8, K), jnp.int32),
        pltpu.VMEM((8, K), jnp.int32),
        pltpu.VMEM((2 * K, H), jnp.float32),
        pltpu.VMEM_SHARED((NP, H), jnp.float32),
        [pltpu.SemaphoreType.DMA] * NQ,
        [pltpu.SemaphoreType.DMA] * NQ,
        pltpu.SemaphoreType.DMA,
        pltpu.SemaphoreType.DMA,
    ],
)
def _scatter_kernel(hn_hbm, pidx_hbm, out_hbm,
                    pk_v, sring, dring, rows_v, spagg, g0, g1, s0, s1):
    c = lax.axis_index("c")
    s = lax.axis_index("s")
    w = c * NS + s
    QR = K // NQ

    def _gather(slot, gsems):
        for q in range(NQ):
            pltpu.async_copy(
                hn_hbm.at[sring.at[slot, pl.ds(q * QR, QR)]],
                rows_v.at[pl.ds(slot * K + q * QR, QR)], gsems[q])

    def _wait_gather(gsems):
        for q in range(NQ):
            pltpu.make_async_copy(hn_hbm.at[pl.ds(0, QR)],
                                  rows_v.at[pl.ds(0, QR)], gsems[q]).wait()

    def _wait64k(sem):
        pltpu.make_async_copy(hn_hbm.at[pl.ds(0, K)],
                              rows_v.at[pl.ds(0, K)], sem).wait()

    def _unpack_src(j, slot):
        for tt in range(K // 16):
            v = pk_v[j, pl.ds(tt * 16, 16)]
            sring[slot, pl.ds(tt * 16, 16)] = jnp.bitwise_and(v, 0xFFFF)

    def _unpack_dst(j, slot):
        for tt in range(K // 16):
            v = pk_v[j, pl.ds(tt * 16, 16)]
            dring[slot, pl.ds(tt * 16, 16)] = lax.shift_right_logical(v, 16)

    _zero_rows(rows_v, K)
    _zero_shared(rows_v, spagg, s)

    pltpu.sync_copy(pidx_hbm.at[w], pk_v)
    _unpack_src(0, 0)
    _unpack_dst(0, 0)
    _unpack_src(1, 1)
    _unpack_dst(1, 1)
    plsc.subcore_barrier()

    _gather(0, g0)
    _gather(1, g1)

    def _body(t, carry):
        g = 2 * t
        _wait_gather(g0)
        pltpu.async_copy(rows_v.at[pl.ds(0, K)], spagg.at[dring.at[0]], s0,
                         add=True)
        _wait_gather(g1)
        pltpu.async_copy(rows_v.at[pl.ds(K, K)], spagg.at[dring.at[1]], s1,
                         add=True)
        _unpack_src(g + 2, 0)
        _wait64k(s0)
        _unpack_dst(g + 2, 0)
        _gather(0, g0)
        _unpack_src(g + 3, 1)
        _wait64k(s1)
        _unpack_dst(g + 3, 1)
        _gather(1, g1)
        return carry

    lax.fori_loop(0, (CH - 2) // 2, _body, 0)

    _wait_gather(g0)
    pltpu.sync_copy(rows_v.at[pl.ds(0, K)], spagg.at[dring.at[0]], add=True)
    _wait_gather(g1)
    pltpu.sync_copy(rows_v.at[pl.ds(K, K)], spagg.at[dring.at[1]], add=True)
    plsc.subcore_barrier()

    _copy_out(spagg, out_hbm, c, s)


def _mm1_body(deg_ref, x_ref, w_ref, out_ref):
    d = deg_ref[0, :, 0:1] + deg_ref[1, :, 0:1] + 1.0
    dinv = lax.rsqrt(d)
    h = jnp.dot(x_ref[...], w_ref[...], preferred_element_type=jnp.float32)
    out_ref[...] = dinv * h


_mm1 = pl.pallas_call(
    _mm1_body,
    grid=(NB,),
    in_specs=[
        pl.BlockSpec((NC, R, H), lambda i: (0, i, 0)),
        pl.BlockSpec((R, D), lambda i: (i, 0)),
        pl.BlockSpec((D, H), lambda i: (0, 0)),
    ],
    out_specs=pl.BlockSpec((R, H), lambda i: (i, 0)),
    out_shape=jax.ShapeDtypeStruct((N, H), jnp.float32),
)


def _mm2_body(deg_ref, p_ref, hn_ref, b_ref, w_ref, out_ref):
    d = deg_ref[0, :, 0:1] + deg_ref[1, :, 0:1] + 1.0
    dinv = lax.rsqrt(d)
    a = jnp.maximum(dinv * (p_ref[0] + p_ref[1] + hn_ref[...]) + b_ref[...],
                    0.0)
    out_ref[...] = dinv * jnp.dot(a, w_ref[...],
                                  preferred_element_type=jnp.float32)


_mm2 = pl.pallas_call(
    _mm2_body,
    grid=(NB,),
    in_specs=[
        pl.BlockSpec((NC, R, H), lambda i: (0, i, 0)),
        pl.BlockSpec((NC, R, H), lambda i: (0, i, 0)),
        pl.BlockSpec((R, H), lambda i: (i, 0)),
        pl.BlockSpec((1, H), lambda i: (0, 0)),
        pl.BlockSpec((H, H), lambda i: (0, 0)),
    ],
    out_specs=pl.BlockSpec((R, H), lambda i: (i, 0)),
    out_shape=jax.ShapeDtypeStruct((N, H), jnp.float32),
)


def _head_body(deg_ref, p_ref, hn_ref, b2_ref, batch_ref, wih_ref, bih_ref,
               bhh_ref, wfc_ref, bfc_ref, out_ref, acc_ref, cnt_ref):
    i = pl.program_id(0)

    @pl.when(i == 0)
    def _init():
        acc_ref[...] = jnp.zeros_like(acc_ref)
        cnt_ref[...] = jnp.zeros_like(cnt_ref)

    d = deg_ref[0, :, 0:1] + deg_ref[1, :, 0:1] + 1.0
    dinv = lax.rsqrt(d)
    h2 = jnp.maximum(dinv * (p_ref[0] + p_ref[1] + hn_ref[...]) + b2_ref[...],
                     0.0)
    b = batch_ref[0]
    gid = lax.broadcasted_iota(jnp.int32, (G, 1), 0)
    mask = (b == gid).astype(jnp.float32)
    acc_ref[...] += jnp.dot(mask, h2, preferred_element_type=jnp.float32)
    cnt_ref[...] += jnp.sum(mask, axis=1, keepdims=True)

    @pl.when(i == NB - 1)
    def _final():
        pooled = acc_ref[...] / jnp.maximum(cnt_ref[...], 1.0)
        gates = lax.dot_general(
            pooled, wih_ref[...], (((1,), (1,)), ((), ())),
            preferred_element_type=jnp.float32,
        ) + bih_ref[...] + bhh_ref[...]
        gi = gates[:, 0:LH]
        gg = gates[:, 2 * LH:3 * LH]
        go = gates[:, 3 * LH:4 * LH]
        cc = jax.nn.sigmoid(gi) * jnp.tanh(gg)
        hh = jax.nn.sigmoid(go) * jnp.tanh(cc)
        out_ref[...] = lax.dot_general(
            hh, wfc_ref[...], (((1,), (1,)), ((), ())),
            preferred_element_type=jnp.float32,
        ) + bfc_ref[...]


_head = pl.pallas_call(
    _head_body,
    grid=(NB,),
    in_specs=[
        pl.BlockSpec((NC, R, H), lambda i: (0, i, 0)),
        pl.BlockSpec((NC, R, H), lambda i: (0, i, 0)),
        pl.BlockSpec((R, H), lambda i: (i, 0)),
        pl.BlockSpec((1, H), lambda i: (0, 0)),
        pl.BlockSpec((1, 1, R), lambda i: (i, 0, 0)),
        pl.BlockSpec((4 * LH, H), lambda i: (0, 0)),
        pl.BlockSpec((1, 4 * LH), lambda i: (0, 0)),
        pl.BlockSpec((1, 4 * LH), lambda i: (0, 0)),
        pl.BlockSpec((C, LH), lambda i: (0, 0)),
        pl.BlockSpec((1, C), lambda i: (0, 0)),
    ],
    out_specs=pl.BlockSpec((G, C), lambda i: (0, 0)),
    out_shape=jax.ShapeDtypeStruct((G, C), jnp.float32),
    scratch_shapes=[
        pltpu.VMEM((G, LH), jnp.float32),
        pltpu.VMEM((G, 1), jnp.float32),
    ],
)


def _pad_edges(idx, fill):
    per_w = idx.reshape(NW, EPW)
    pad = jnp.full((NW, PADW), fill, dtype=idx.dtype)
    return jnp.concatenate([per_w, pad], axis=1).reshape(NW, CH, K)


def kernel(x, edge_index, batch, W1, b1, W2, b2, W_ih, W_hh, b_ih, b_hh,
           W_fc, b_fc):
    src = _pad_edges(edge_index[0], 0)
    dst = _pad_edges(edge_index[1], N)
    pidx = jnp.bitwise_or(src, jnp.left_shift(dst, 16))

    degp = _deg_kernel(dst)
    hn1 = _mm1(degp, x, W1)
    part1 = _scatter_kernel(hn1, pidx)
    hn2 = _mm2(degp, part1, hn1, b1.reshape(1, H), W2)
    part2 = _scatter_kernel(hn2, pidx)
    out = _head(degp, part2, hn2, b2.reshape(1, H),
                batch.reshape(NB, 1, R), W_ih, b_ih.reshape(1, 4 * LH),
                b_hh.reshape(1, 4 * LH), W_fc, b_fc.reshape(1, C))
    return out
